# baseline (speedup 1.0000x reference)
import jax
import jax.numpy as jnp
from jax.experimental import pallas as pl
from jax.experimental.pallas import tpu as pltpu


def _band_weights(blk, nblocks, shift, k):
    m0, kb = blk.shape
    padw = k + shift
    d = jnp.zeros((m0, nblocks, padw), blk.dtype)
    d = d.at[:, :, :kb].set(blk[:, None, :])
    flat = d.reshape(m0, nblocks * padw)[:, : nblocks * k]
    w = flat.reshape(m0, nblocks, k)
    return jnp.transpose(w, (1, 0, 2)).reshape(nblocks * m0, k)


def _cnn01_fused_kernel(xs_ref, w1a_ref, w1b_ref, w2g_ref, wfc_ref, b_ref, o_ref):
    tb = xs_ref.shape[-1]
    f32 = jnp.float32

    a1a = jnp.dot(w1a_ref[...], xs_ref[0:200, :], preferred_element_type=f32)
    a1b = jnp.dot(w1b_ref[...], xs_ref[192:384, :], preferred_element_type=f32)
    a1 = jnp.concatenate([a1a, a1b], axis=0)

    a1 = a1.reshape(63, 32, tb)
    pooled = jnp.maximum(jnp.maximum(a1[:, :16, :], a1[:, 16:, :]), 0.0)
    pooled = pooled.reshape(1008, tb)
    pooled = jnp.concatenate([pooled, jnp.zeros((64, tb), f32)], axis=0)

    w2g = w2g_ref[...]
    acc = b_ref[...]
    for g in range(5):
        p = pooled[208 * g: 208 * g + 240, :]
        a2 = jnp.dot(w2g, p, preferred_element_type=f32)
        a2 = jnp.maximum(a2, 0.0)
        acc = acc + jnp.dot(wfc_ref[:, 416 * g: 416 * g + 416], a2,
                            preferred_element_type=f32)
    o_ref[...] = acc


def kernel(x, w1, w2, wfc, bfc):
    N, C, H, W = x.shape
    assert C == 1 and W == 3 and H == 128
    H2 = 61
    out_size = bfc.shape[0]
    f32 = jnp.float32

    TB = 4096
    Npad = pl.cdiv(N, TB) * TB

    xs = x[:, 0, :, :].reshape(N, H * W).astype(f32).T
    if Npad != N:
        xs = jnp.pad(xs, ((0, 0), (0, Npad - N)))

    w1blk = w1[:, 0, :, :].reshape(16, 9).astype(f32)
    w1a = _band_weights(w1blk, 64, 3, 200)
    w1b = _band_weights(w1blk, 62, 3, 192)

    w2blk = jnp.transpose(w2[:, :, :, 0], (0, 2, 1)).reshape(32, 48)
    w2g = _band_weights(w2blk.astype(f32), 13, 16, 240)

    wfc3 = wfc.reshape(out_size, 32, H2).astype(f32)
    wfc3 = jnp.pad(wfc3, ((0, 0), (0, 0), (0, 65 - H2)))
    wfc_r = jnp.transpose(wfc3, (0, 2, 1)).reshape(out_size, 65 * 32)
    bfc_r = bfc.reshape(out_size, 1).astype(f32)

    y = pl.pallas_call(
        _cnn01_fused_kernel,
        out_shape=jax.ShapeDtypeStruct((out_size, Npad), jnp.float32),
        grid=(Npad // TB,),
        in_specs=[
            pl.BlockSpec((H * W, TB), lambda g: (0, g)),
            pl.BlockSpec((1024, 200), lambda g: (0, 0)),
            pl.BlockSpec((992, 192), lambda g: (0, 0)),
            pl.BlockSpec((416, 240), lambda g: (0, 0)),
            pl.BlockSpec((out_size, 2080), lambda g: (0, 0)),
            pl.BlockSpec((out_size, 1), lambda g: (0, 0)),
        ],
        out_specs=pl.BlockSpec((out_size, TB), lambda g: (0, g)),
        compiler_params=pltpu.CompilerParams(dimension_semantics=("parallel",)),
    )(xs, w1a, w1b, w2g, wfc_r, bfc_r)
    return y.T[:N]


# speedup vs baseline: 1.0020x; 1.0020x over previous
import jax
import jax.numpy as jnp
from jax.experimental import pallas as pl
from jax.experimental.pallas import tpu as pltpu


def _band_weights(blk, nblocks, shift, k):
    m0, kb = blk.shape
    padw = k + shift
    d = jnp.zeros((m0, nblocks, padw), blk.dtype)
    d = d.at[:, :, :kb].set(blk[:, None, :])
    flat = d.reshape(m0, nblocks * padw)[:, : nblocks * k]
    w = flat.reshape(m0, nblocks, k)
    return jnp.transpose(w, (1, 0, 2)).reshape(nblocks * m0, k)


def _cnn01_fused_kernel(xs_ref, w1a_ref, w1b_ref, w2g_ref, wfc_ref, b_ref, o_ref):
    tb = xs_ref.shape[-1]
    f32 = jnp.float32

    a1a = jnp.dot(w1a_ref[...], xs_ref[0:200, :], preferred_element_type=f32)
    a1b = jnp.dot(w1b_ref[...], xs_ref[192:384, :], preferred_element_type=f32)
    a1 = jnp.concatenate([a1a, a1b], axis=0)

    a1 = a1.reshape(63, 32, tb)
    pooled = jnp.maximum(jnp.maximum(a1[:, :16, :], a1[:, 16:, :]), 0.0)
    pooled = pooled.reshape(1008, tb)
    pooled = jnp.concatenate([pooled, jnp.zeros((64, tb), f32)], axis=0)

    w2g = w2g_ref[...]
    acc = b_ref[...]
    for g in range(5):
        p = pooled[208 * g: 208 * g + 240, :]
        a2 = jnp.dot(w2g, p, preferred_element_type=f32)
        a2 = jnp.maximum(a2, 0.0)
        acc = acc + jnp.dot(wfc_ref[:, 416 * g: 416 * g + 416], a2,
                            preferred_element_type=f32)
    o_ref[...] = acc


def kernel(x, w1, w2, wfc, bfc):
    N, C, H, W = x.shape
    assert C == 1 and W == 3 and H == 128
    H2 = 61
    out_size = bfc.shape[0]
    f32 = jnp.float32

    TB = 2048
    Npad = pl.cdiv(N, TB) * TB

    xs = x[:, 0, :, :].reshape(N, H * W).astype(f32).T
    if Npad != N:
        xs = jnp.pad(xs, ((0, 0), (0, Npad - N)))

    w1blk = w1[:, 0, :, :].reshape(16, 9).astype(f32)
    w1a = _band_weights(w1blk, 64, 3, 200)
    w1b = _band_weights(w1blk, 62, 3, 192)

    w2blk = jnp.transpose(w2[:, :, :, 0], (0, 2, 1)).reshape(32, 48)
    w2g = _band_weights(w2blk.astype(f32), 13, 16, 240)

    wfc3 = wfc.reshape(out_size, 32, H2).astype(f32)
    wfc3 = jnp.pad(wfc3, ((0, 0), (0, 0), (0, 65 - H2)))
    wfc_r = jnp.transpose(wfc3, (0, 2, 1)).reshape(out_size, 65 * 32)
    bfc_r = bfc.reshape(out_size, 1).astype(f32)

    y = pl.pallas_call(
        _cnn01_fused_kernel,
        out_shape=jax.ShapeDtypeStruct((out_size, Npad), jnp.float32),
        grid=(Npad // TB,),
        in_specs=[
            pl.BlockSpec((H * W, TB), lambda g: (0, g)),
            pl.BlockSpec((1024, 200), lambda g: (0, 0)),
            pl.BlockSpec((992, 192), lambda g: (0, 0)),
            pl.BlockSpec((416, 240), lambda g: (0, 0)),
            pl.BlockSpec((out_size, 2080), lambda g: (0, 0)),
            pl.BlockSpec((out_size, 1), lambda g: (0, 0)),
        ],
        out_specs=pl.BlockSpec((out_size, TB), lambda g: (0, g)),
        compiler_params=pltpu.CompilerParams(dimension_semantics=("parallel",)),
    )(xs, w1a, w1b, w2g, wfc_r, bfc_r)
    return y.T[:N]


# speedup vs baseline: 1.0705x; 1.0684x over previous
import jax
import jax.numpy as jnp
from jax.experimental import pallas as pl
from jax.experimental.pallas import tpu as pltpu


def _band_weights(blk, nblocks, shift, k):
    m0, kb = blk.shape
    padw = k + shift
    d = jnp.zeros((m0, nblocks, padw), blk.dtype)
    d = d.at[:, :, :kb].set(blk[:, None, :])
    flat = d.reshape(m0, nblocks * padw)[:, : nblocks * k]
    w = flat.reshape(m0, nblocks, k)
    return jnp.transpose(w, (1, 0, 2)).reshape(nblocks * m0, k)


def _cnn01_fused_kernel(xs_ref, w1a_ref, w1b_ref, w2g_ref, wfc_ref, b_ref, o_ref):
    tb = xs_ref.shape[-1]
    f32 = jnp.float32

    a1a = jnp.dot(w1a_ref[...], xs_ref[0:200, :], preferred_element_type=f32)
    a1b = jnp.dot(w1b_ref[...], xs_ref[192:384, :], preferred_element_type=f32)
    a1 = jnp.concatenate([a1a, a1b], axis=0)

    a1 = a1.reshape(63, 32, tb)
    pooled = jnp.maximum(jnp.maximum(a1[:, :16, :], a1[:, 16:, :]), 0.0)
    pooled = pooled.reshape(1008, tb)
    pooled = jnp.concatenate([pooled, jnp.zeros((64, tb), f32)], axis=0)

    w2g = w2g_ref[...]
    acc = b_ref[...]
    for g in range(5):
        p = pooled[208 * g: 208 * g + 240, :]
        a2 = jnp.dot(w2g, p, preferred_element_type=f32)
        a2 = jnp.maximum(a2, 0.0)
        acc = acc + jnp.dot(wfc_ref[:, 416 * g: 416 * g + 416], a2,
                            preferred_element_type=f32)
    o_ref[...] = acc.T


def kernel(x, w1, w2, wfc, bfc):
    N, C, H, W = x.shape
    assert C == 1 and W == 3 and H == 128
    H2 = 61
    out_size = bfc.shape[0]
    f32 = jnp.float32

    TB = 2048
    Npad = pl.cdiv(N, TB) * TB

    xs = x[:, 0, :, :].reshape(N, H * W).astype(f32).T
    if Npad != N:
        xs = jnp.pad(xs, ((0, 0), (0, Npad - N)))

    w1blk = w1[:, 0, :, :].reshape(16, 9).astype(f32)
    w1a = _band_weights(w1blk, 64, 3, 200)
    w1b = _band_weights(w1blk, 62, 3, 192)

    w2blk = jnp.transpose(w2[:, :, :, 0], (0, 2, 1)).reshape(32, 48)
    w2g = _band_weights(w2blk.astype(f32), 13, 16, 240)

    wfc3 = wfc.reshape(out_size, 32, H2).astype(f32)
    wfc3 = jnp.pad(wfc3, ((0, 0), (0, 0), (0, 65 - H2)))
    wfc_r = jnp.transpose(wfc3, (0, 2, 1)).reshape(out_size, 65 * 32)
    bfc_r = bfc.reshape(out_size, 1).astype(f32)

    y = pl.pallas_call(
        _cnn01_fused_kernel,
        out_shape=jax.ShapeDtypeStruct((Npad, out_size), jnp.float32),
        grid=(Npad // TB,),
        in_specs=[
            pl.BlockSpec((H * W, TB), lambda g: (0, g)),
            pl.BlockSpec((1024, 200), lambda g: (0, 0)),
            pl.BlockSpec((992, 192), lambda g: (0, 0)),
            pl.BlockSpec((416, 240), lambda g: (0, 0)),
            pl.BlockSpec((out_size, 2080), lambda g: (0, 0)),
            pl.BlockSpec((out_size, 1), lambda g: (0, 0)),
        ],
        out_specs=pl.BlockSpec((TB, out_size), lambda g: (g, 0)),
        compiler_params=pltpu.CompilerParams(dimension_semantics=("parallel",)),
    )(xs, w1a, w1b, w2g, wfc_r, bfc_r)
    return y[:N]
